# baseline (speedup 1.0000x reference)
import functools

import jax
import jax.numpy as jnp
from jax import lax
from jax.experimental import pallas as pl
from jax.experimental.pallas import tpu as pltpu
from jax.experimental.pallas import tpu_sc as plsc

_T = 2048
_D = 768
_E = 64
_F = 64
_PT = 256
_ST = 128
_NW = 32
_CHUNK = _T // _NW

_DN_T = (((1,), (1,)), ((), ()))


def _plan_kernel(x_ref, rb_ref, routert_ref, dst_ref, se_ref, bounds_ref,
                 p_ref):
    x = x_ref[...]
    logits = jax.lax.dot_general(x - rb_ref[...], routert_ref[...], _DN_T,
                                 preferred_element_type=jnp.float32)
    m = jnp.max(logits, axis=-1, keepdims=True)
    z = jnp.sum(jnp.exp(logits - m), axis=-1, keepdims=True)
    idx = jnp.argmax(logits, axis=-1)

    onehot = (jax.lax.broadcasted_iota(jnp.int32, (_T, _E), 1)
              == idx[:, None]).astype(jnp.float32)

    tri = (jax.lax.broadcasted_iota(jnp.int32, (_PT, _PT), 0)
           > jax.lax.broadcasted_iota(jnp.int32, (_PT, _PT), 1)
           ).astype(jnp.float32)
    ones_row = jnp.ones((1, _PT), dtype=jnp.float32)

    counts = jnp.zeros((1, _E), jnp.float32)
    rank_tiles = []
    for i in range(_T // _PT):
        blk = onehot[i * _PT:(i + 1) * _PT, :]
        rank_tiles.append(
            jnp.dot(tri, blk, preferred_element_type=jnp.float32) + counts)
        counts = counts + jnp.dot(ones_row, blk,
                                  preferred_element_type=jnp.float32)
    rank_all = jnp.concatenate(rank_tiles, axis=0)
    rank = jnp.sum(rank_all * onehot, axis=-1, keepdims=True)

    lt = (jax.lax.broadcasted_iota(jnp.int32, (_E, _E), 0)
          < jax.lax.broadcasted_iota(jnp.int32, (_E, _E), 1)).astype(jnp.float32)
    offsets = jnp.dot(counts, lt, preferred_element_type=jnp.float32)
    off_tok = jnp.sum(onehot * offsets, axis=-1, keepdims=True)

    cum_incl = offsets + counts
    slot = jax.lax.broadcasted_iota(jnp.int32, (_T, 1), 0).astype(jnp.float32)
    se = jnp.sum((cum_incl <= slot).astype(jnp.int32), axis=-1, keepdims=True)

    dst_ref[...] = (rank + off_tok).astype(jnp.int32).reshape(_T)
    se_ref[...] = se.reshape(_T)
    pad = jnp.zeros((1, 128 - _E), jnp.float32)
    bounds_ref[...] = jnp.concatenate(
        [offsets, jnp.full((1, 1), float(_T), jnp.float32), pad[:, 1:]],
        axis=1).astype(jnp.int32).reshape(128)
    p_ref[...] = (1.0 / z).reshape(_T)


def _plan(activations, router_b, router_t):
    return pl.pallas_call(
        _plan_kernel,
        in_specs=[
            pl.BlockSpec((_T, _D), lambda: (0, 0)),
            pl.BlockSpec((1, _D), lambda: (0, 0)),
            pl.BlockSpec((_E, _D), lambda: (0, 0)),
        ],
        out_specs=[
            pl.BlockSpec((_T,), lambda: (0,)),
            pl.BlockSpec((_T,), lambda: (0,)),
            pl.BlockSpec((128,), lambda: (0,)),
            pl.BlockSpec((_T,), lambda: (0,)),
        ],
        out_shape=[
            jax.ShapeDtypeStruct((_T,), jnp.int32),
            jax.ShapeDtypeStruct((_T,), jnp.int32),
            jax.ShapeDtypeStruct((128,), jnp.int32),
            jax.ShapeDtypeStruct((_T,), jnp.float32),
        ],
    )(activations, router_b.reshape(1, _D), router_t)


def _inv_kernel(dst_ref, src_ref):
    def body(t, carry):
        src_ref[dst_ref[t]] = t
        return carry

    jax.lax.fori_loop(0, _T, body, 0, unroll=8)


def _invert(dst):
    return pl.pallas_call(
        _inv_kernel,
        in_specs=[pl.BlockSpec(memory_space=pltpu.SMEM)],
        out_specs=pl.BlockSpec(memory_space=pltpu.SMEM),
        out_shape=jax.ShapeDtypeStruct((_T,), jnp.int32),
    )(dst)


def _sc_gather_body(x_hbm, p_hbm, src_hbm, xs_hbm, ps_hbm,
                    src_v, rows_v, p_v, sem, sem2):
    wid = lax.axis_index("s") * 2 + lax.axis_index("c")
    base = wid * _CHUNK
    pltpu.sync_copy(src_hbm.at[pl.ds(base, _CHUNK)], src_v)
    c1 = pltpu.async_copy(x_hbm.at[src_v], rows_v, sem)
    c2 = pltpu.async_copy(p_hbm.at[src_v], p_v, sem2)
    c1.wait()
    c2.wait()
    d1 = pltpu.async_copy(rows_v, xs_hbm.at[pl.ds(base, _CHUNK)], sem)
    d2 = pltpu.async_copy(p_v, ps_hbm.at[pl.ds(base, _CHUNK)], sem2)
    d1.wait()
    d2.wait()


def _sc_gather(x, p, src):
    mesh = plsc.VectorSubcoreMesh(core_axis_name="c", subcore_axis_name="s")
    f = functools.partial(
        pl.kernel, mesh=mesh,
        out_type=[
            jax.ShapeDtypeStruct((_T, _D), jnp.float32),
            jax.ShapeDtypeStruct((_T,), jnp.float32),
        ],
        scratch_types=[
            pltpu.VMEM((_CHUNK,), jnp.int32),
            pltpu.VMEM((_CHUNK, _D), jnp.float32),
            pltpu.VMEM((_CHUNK,), jnp.float32),
            pltpu.SemaphoreType.DMA,
            pltpu.SemaphoreType.DMA,
        ],
    )(_sc_gather_body)
    return f(x, p, src)


_SLOTS = 6


def _seg_kernel(se_smem, bounds_smem, xs_ref, bpre_ref, enct_ref, dec_ref,
                ps_ref, out_ref):
    t = pl.program_id(0)
    e_lo = se_smem[t * _ST]
    e_hi = se_smem[t * _ST + _ST - 1]
    a = (xs_ref[...] - bpre_ref[...]).astype(jnp.bfloat16)
    grow = jax.lax.broadcasted_iota(jnp.int32, (_ST, 1), 0) + t * _ST

    def one_expert(e, acc, valid=True):
        lat = jax.lax.dot_general(a, enct_ref[e], _DN_T,
                                  preferred_element_type=jnp.float32)
        lat = jnp.maximum(lat, 0.0)
        seg_mask = (grow >= bounds_smem[e]) & (grow < bounds_smem[e + 1])
        lat = jnp.where(seg_mask & valid, lat, 0.0).astype(jnp.bfloat16)
        return acc + jnp.dot(lat, dec_ref[e], preferred_element_type=jnp.float32)

    acc = jnp.zeros((_ST, _D), jnp.float32)
    for i in range(_SLOTS):
        e = jnp.minimum(e_lo + i, _E - 1)
        acc = one_expert(e, acc, valid=(e_lo + i) <= e_hi)

    acc = jax.lax.fori_loop(
        _SLOTS, e_hi - e_lo + 1,
        lambda i, s: one_expert(e_lo + i, s), acc)

    ps_col = jnp.transpose(ps_ref[...], (0, 2, 1)).reshape(_ST, 1)
    out_ref[...] = ps_col * acc + bpre_ref[...]


def _segment(xs, se, bounds, ps, b_pre, enc_t, dec):
    grid_spec = pltpu.PrefetchScalarGridSpec(
        num_scalar_prefetch=2,
        grid=(_T // _ST,),
        in_specs=[
            pl.BlockSpec((_ST, _D), lambda t, se, b: (t, 0)),
            pl.BlockSpec((1, _D), lambda t, se, b: (0, 0)),
            pl.BlockSpec((_E, _F, _D), lambda t, se, b: (0, 0, 0)),
            pl.BlockSpec((_E, _F, _D), lambda t, se, b: (0, 0, 0)),
            pl.BlockSpec((1, 1, _ST), lambda t, se, b: (t, 0, 0)),
        ],
        out_specs=pl.BlockSpec((_ST, _D), lambda t, se, b: (t, 0)),
    )
    return pl.pallas_call(
        _seg_kernel,
        grid_spec=grid_spec,
        out_shape=jax.ShapeDtypeStruct((_T, _D), jnp.float32),
    )(se, bounds, xs, b_pre.reshape(1, _D), enc_t, dec,
      ps.reshape(_T // _ST, 1, _ST))


def _sc_unsort_body(ys_hbm, dst_hbm, out_hbm, dst_v, rows_v, sem):
    wid = lax.axis_index("s") * 2 + lax.axis_index("c")
    base = wid * _CHUNK
    pltpu.sync_copy(dst_hbm.at[pl.ds(base, _CHUNK)], dst_v)
    pltpu.async_copy(ys_hbm.at[dst_v], rows_v, sem).wait()
    pltpu.sync_copy(rows_v, out_hbm.at[pl.ds(base, _CHUNK)])


def _sc_unsort(ys, dst):
    mesh = plsc.VectorSubcoreMesh(core_axis_name="c", subcore_axis_name="s")
    f = functools.partial(
        pl.kernel, mesh=mesh,
        out_type=jax.ShapeDtypeStruct((_T, _D), jnp.float32),
        scratch_types=[
            pltpu.VMEM((_CHUNK,), jnp.int32),
            pltpu.VMEM((_CHUNK, _D), jnp.float32),
            pltpu.SemaphoreType.DMA,
        ],
    )(_sc_unsort_body)
    return f(ys, dst)


def kernel(activations, b_pre, enc, dec, router_b, router):
    router_t = router.T
    enc_t = enc.transpose(0, 2, 1)
    dst, se, bounds, p = _plan(activations, router_b, router_t)
    src = _invert(dst)
    xs, ps = _sc_gather(activations, p, src)
    ys = _segment(xs, se, bounds, ps, b_pre,
                  enc_t.astype(jnp.bfloat16), dec.astype(jnp.bfloat16))
    return _sc_unsort(ys, dst)


# speedup vs baseline: 1.1165x; 1.1165x over previous
import functools

import jax
import jax.numpy as jnp
from jax import lax
from jax.experimental import pallas as pl
from jax.experimental.pallas import tpu as pltpu
from jax.experimental.pallas import tpu_sc as plsc

_T = 2048
_D = 768
_E = 64
_F = 64
_PT = 256
_ST = 128
_NW = 32
_CHUNK = _T // _NW

_DN_T = (((1,), (1,)), ((), ()))


def _plan_kernel(x_ref, rb_ref, routert_ref, dst_ref, bounds_ref, p_ref):
    x = x_ref[...]
    logits = jax.lax.dot_general(x - rb_ref[...], routert_ref[...], _DN_T,
                                 preferred_element_type=jnp.float32)
    m = jnp.max(logits, axis=-1, keepdims=True)
    z = jnp.sum(jnp.exp(logits - m), axis=-1, keepdims=True)
    idx = jnp.argmax(logits, axis=-1)

    onehot = (jax.lax.broadcasted_iota(jnp.int32, (_T, _E), 1)
              == idx[:, None]).astype(jnp.float32)

    tri = (jax.lax.broadcasted_iota(jnp.int32, (_PT, _PT), 0)
           > jax.lax.broadcasted_iota(jnp.int32, (_PT, _PT), 1)
           ).astype(jnp.float32)
    ones_row = jnp.ones((1, _PT), dtype=jnp.float32)

    counts = jnp.zeros((1, _E), jnp.float32)
    rank_tiles = []
    for i in range(_T // _PT):
        blk = onehot[i * _PT:(i + 1) * _PT, :]
        rank_tiles.append(
            jnp.dot(tri, blk, preferred_element_type=jnp.float32) + counts)
        counts = counts + jnp.dot(ones_row, blk,
                                  preferred_element_type=jnp.float32)
    rank_all = jnp.concatenate(rank_tiles, axis=0)
    rank = jnp.sum(rank_all * onehot, axis=-1, keepdims=True)

    lt = (jax.lax.broadcasted_iota(jnp.int32, (_E, _E), 0)
          < jax.lax.broadcasted_iota(jnp.int32, (_E, _E), 1)).astype(jnp.float32)
    offsets = jnp.dot(counts, lt, preferred_element_type=jnp.float32)
    off_tok = jnp.sum(onehot * offsets, axis=-1, keepdims=True)

    cum_incl = offsets + counts
    cum_col = jnp.transpose(cum_incl, (1, 0))
    nt = _T // _ST
    slot_lo = (jax.lax.broadcasted_iota(jnp.int32, (1, nt), 1)
               * _ST).astype(jnp.float32)
    tile_lo = jnp.sum((cum_col <= slot_lo).astype(jnp.int32), axis=0,
                      keepdims=True)
    tile_hi = jnp.sum((cum_col <= slot_lo + float(_ST - 1)).astype(jnp.int32),
                      axis=0, keepdims=True)

    dst_ref[...] = (rank + off_tok).astype(jnp.int32).reshape(_T)
    bounds_ref[...] = jnp.concatenate(
        [offsets.astype(jnp.int32),
         jnp.full((1, 1), _T, jnp.int32),
         jnp.zeros((1, 127 - _E - 2 * nt), jnp.int32),
         tile_lo, tile_hi], axis=1).reshape(128)
    p_ref[...] = (1.0 / z).reshape(_T)


def _plan(activations, router_b, router_t):
    return pl.pallas_call(
        _plan_kernel,
        in_specs=[
            pl.BlockSpec((_T, _D), lambda: (0, 0)),
            pl.BlockSpec((1, _D), lambda: (0, 0)),
            pl.BlockSpec((_E, _D), lambda: (0, 0)),
        ],
        out_specs=[
            pl.BlockSpec((_T,), lambda: (0,)),
            pl.BlockSpec((128,), lambda: (0,)),
            pl.BlockSpec((_T,), lambda: (0,)),
        ],
        out_shape=[
            jax.ShapeDtypeStruct((_T,), jnp.int32),
            jax.ShapeDtypeStruct((128,), jnp.int32),
            jax.ShapeDtypeStruct((_T,), jnp.float32),
        ],
    )(activations, router_b.reshape(1, _D), router_t)


def _inv_kernel(dst_ref, src_ref):
    def body(t, carry):
        src_ref[dst_ref[t]] = t
        return carry

    jax.lax.fori_loop(0, _T, body, 0, unroll=8)


def _invert(dst):
    return pl.pallas_call(
        _inv_kernel,
        in_specs=[pl.BlockSpec(memory_space=pltpu.SMEM)],
        out_specs=pl.BlockSpec(memory_space=pltpu.SMEM),
        out_shape=jax.ShapeDtypeStruct((_T,), jnp.int32),
    )(dst)


def _sc_gather_body(x_hbm, p_hbm, src_hbm, xs_hbm, ps_hbm,
                    src_v, rows_v, p_v, sem, sem2):
    wid = lax.axis_index("s") * 2 + lax.axis_index("c")
    base = wid * _CHUNK
    pltpu.sync_copy(src_hbm.at[pl.ds(base, _CHUNK)], src_v)
    c1 = pltpu.async_copy(x_hbm.at[src_v], rows_v, sem)
    c2 = pltpu.async_copy(p_hbm.at[src_v], p_v, sem2)
    c1.wait()
    c2.wait()
    d1 = pltpu.async_copy(rows_v, xs_hbm.at[pl.ds(base, _CHUNK)], sem)
    d2 = pltpu.async_copy(p_v, ps_hbm.at[pl.ds(base, _CHUNK)], sem2)
    d1.wait()
    d2.wait()


def _sc_gather(x, p, src):
    mesh = plsc.VectorSubcoreMesh(core_axis_name="c", subcore_axis_name="s")
    f = functools.partial(
        pl.kernel, mesh=mesh,
        out_type=[
            jax.ShapeDtypeStruct((_T, _D), jnp.float32),
            jax.ShapeDtypeStruct((_T,), jnp.float32),
        ],
        scratch_types=[
            pltpu.VMEM((_CHUNK,), jnp.int32),
            pltpu.VMEM((_CHUNK, _D), jnp.float32),
            pltpu.VMEM((_CHUNK,), jnp.float32),
            pltpu.SemaphoreType.DMA,
            pltpu.SemaphoreType.DMA,
        ],
    )(_sc_gather_body)
    return f(x, p, src)


_SLOTS = 6


def _seg_kernel(bounds_smem, xs_ref, bpre_ref, enct_ref, dec_ref,
                ps_ref, out_ref):
    t = pl.program_id(0)
    nt = _T // _ST
    e_lo = bounds_smem[128 - 2 * nt + t]
    e_hi = bounds_smem[128 - nt + t]
    a = (xs_ref[...] - bpre_ref[...]).astype(jnp.bfloat16)
    grow = jax.lax.broadcasted_iota(jnp.int32, (_ST, 1), 0) + t * _ST

    def enc_mm(e):
        return jax.lax.dot_general(a, enct_ref[e], _DN_T,
                                   preferred_element_type=jnp.float32)

    def mask_dec(e, lat, acc, valid):
        lat = jnp.maximum(lat, 0.0)
        seg_mask = (grow >= bounds_smem[e]) & (grow < bounds_smem[e + 1])
        lat = jnp.where(seg_mask & valid, lat, 0.0).astype(jnp.bfloat16)
        return acc + jnp.dot(lat, dec_ref[e], preferred_element_type=jnp.float32)

    es = [jnp.minimum(e_lo + i, _E - 1) for i in range(_SLOTS)]
    lats = [enc_mm(es[i]) for i in range(_SLOTS)]
    acc = jnp.zeros((_ST, _D), jnp.float32)
    for i in range(_SLOTS):
        acc = mask_dec(es[i], lats[i], acc, (e_lo + i) <= e_hi)

    acc = jax.lax.fori_loop(
        _SLOTS, e_hi - e_lo + 1,
        lambda i, s: mask_dec(e_lo + i, enc_mm(e_lo + i), s, True), acc)

    ps_col = jnp.transpose(ps_ref[...], (0, 2, 1)).reshape(_ST, 1)
    out_ref[...] = ps_col * acc + bpre_ref[...]


def _segment(xs, bounds, ps, b_pre, enc_t, dec):
    grid_spec = pltpu.PrefetchScalarGridSpec(
        num_scalar_prefetch=1,
        grid=(_T // _ST,),
        in_specs=[
            pl.BlockSpec((_ST, _D), lambda t, b: (t, 0)),
            pl.BlockSpec((1, _D), lambda t, b: (0, 0)),
            pl.BlockSpec((_E, _F, _D), lambda t, b: (0, 0, 0)),
            pl.BlockSpec((_E, _F, _D), lambda t, b: (0, 0, 0)),
            pl.BlockSpec((1, 1, _ST), lambda t, b: (t, 0, 0)),
        ],
        out_specs=pl.BlockSpec((_ST, _D), lambda t, b: (t, 0)),
    )
    return pl.pallas_call(
        _seg_kernel,
        grid_spec=grid_spec,
        out_shape=jax.ShapeDtypeStruct((_T, _D), jnp.float32),
    )(bounds, xs, b_pre.reshape(1, _D), enc_t, dec,
      ps.reshape(_T // _ST, 1, _ST))


def _sc_unsort_body(ys_hbm, dst_hbm, out_hbm, dst_v, rows_v, sem):
    wid = lax.axis_index("s") * 2 + lax.axis_index("c")
    base = wid * _CHUNK
    pltpu.sync_copy(dst_hbm.at[pl.ds(base, _CHUNK)], dst_v)
    pltpu.async_copy(ys_hbm.at[dst_v], rows_v, sem).wait()
    pltpu.sync_copy(rows_v, out_hbm.at[pl.ds(base, _CHUNK)])


def _sc_unsort(ys, dst):
    mesh = plsc.VectorSubcoreMesh(core_axis_name="c", subcore_axis_name="s")
    f = functools.partial(
        pl.kernel, mesh=mesh,
        out_type=jax.ShapeDtypeStruct((_T, _D), jnp.float32),
        scratch_types=[
            pltpu.VMEM((_CHUNK,), jnp.int32),
            pltpu.VMEM((_CHUNK, _D), jnp.float32),
            pltpu.SemaphoreType.DMA,
        ],
    )(_sc_unsort_body)
    return f(ys, dst)


def kernel(activations, b_pre, enc, dec, router_b, router):
    router_t = router.T
    enc_t = enc.transpose(0, 2, 1)
    dst, bounds, p = _plan(activations, router_b, router_t)
    src = _invert(dst)
    xs, ps = _sc_gather(activations, p, src)
    ys = _segment(xs, bounds, ps, b_pre,
                  enc_t.astype(jnp.bfloat16), dec.astype(jnp.bfloat16))
    return _sc_unsort(ys, dst)


# speedup vs baseline: 1.1360x; 1.0174x over previous
import functools

import jax
import jax.numpy as jnp
from jax import lax
from jax.experimental import pallas as pl
from jax.experimental.pallas import tpu as pltpu
from jax.experimental.pallas import tpu_sc as plsc

_T = 2048
_D = 768
_E = 64
_F = 64
_PT = 256
_ST = 128
_NW = 32
_CHUNK = _T // _NW

_DN_T = (((1,), (1,)), ((), ()))


def _plan_kernel(x_ref, rb_ref, routert_ref, dst_ref, bounds_ref):
    x = x_ref[...]
    logits = jax.lax.dot_general(x - rb_ref[...], routert_ref[...], _DN_T,
                                 preferred_element_type=jnp.float32)
    idx = jnp.argmax(logits, axis=-1)

    onehot = (jax.lax.broadcasted_iota(jnp.int32, (_T, _E), 1)
              == idx[:, None]).astype(jnp.float32)

    tri = (jax.lax.broadcasted_iota(jnp.int32, (_PT, _PT), 0)
           > jax.lax.broadcasted_iota(jnp.int32, (_PT, _PT), 1)
           ).astype(jnp.float32)
    ones_row = jnp.ones((1, _PT), dtype=jnp.float32)

    counts = jnp.zeros((1, _E), jnp.float32)
    rank_tiles = []
    for i in range(_T // _PT):
        blk = onehot[i * _PT:(i + 1) * _PT, :]
        rank_tiles.append(
            jnp.dot(tri, blk, preferred_element_type=jnp.float32) + counts)
        counts = counts + jnp.dot(ones_row, blk,
                                  preferred_element_type=jnp.float32)
    rank_all = jnp.concatenate(rank_tiles, axis=0)
    rank = jnp.sum(rank_all * onehot, axis=-1, keepdims=True)

    lt = (jax.lax.broadcasted_iota(jnp.int32, (_E, _E), 0)
          < jax.lax.broadcasted_iota(jnp.int32, (_E, _E), 1)).astype(jnp.float32)
    offsets = jnp.dot(counts, lt, preferred_element_type=jnp.float32)
    off_tok = jnp.sum(onehot * offsets, axis=-1, keepdims=True)

    cum_incl = offsets + counts
    cum_col = jnp.transpose(cum_incl, (1, 0))
    nt = _T // _ST
    slot_lo = (jax.lax.broadcasted_iota(jnp.int32, (1, nt), 1)
               * _ST).astype(jnp.float32)
    tile_lo = jnp.sum((cum_col <= slot_lo).astype(jnp.int32), axis=0,
                      keepdims=True)
    tile_hi = jnp.sum((cum_col <= slot_lo + float(_ST - 1)).astype(jnp.int32),
                      axis=0, keepdims=True)

    dst_ref[...] = (rank + off_tok).astype(jnp.int32).reshape(_T)
    bounds_ref[...] = jnp.concatenate(
        [offsets.astype(jnp.int32),
         jnp.full((1, 1), _T, jnp.int32),
         jnp.zeros((1, 127 - _E - 2 * nt), jnp.int32),
         tile_lo, tile_hi], axis=1).reshape(128)


def _plan(activations, router_b, router_t):
    return pl.pallas_call(
        _plan_kernel,
        in_specs=[
            pl.BlockSpec((_T, _D), lambda: (0, 0)),
            pl.BlockSpec((1, _D), lambda: (0, 0)),
            pl.BlockSpec((_E, _D), lambda: (0, 0)),
        ],
        out_specs=[
            pl.BlockSpec((_T,), lambda: (0,)),
            pl.BlockSpec((128,), lambda: (0,)),
        ],
        out_shape=[
            jax.ShapeDtypeStruct((_T,), jnp.int32),
            jax.ShapeDtypeStruct((128,), jnp.int32),
        ],
    )(activations, router_b.reshape(1, _D), router_t)


def _inv_kernel(dst_ref, src_ref):
    def body(t, carry):
        src_ref[dst_ref[t]] = t
        return carry

    jax.lax.fori_loop(0, _T, body, 0, unroll=8)


def _invert(dst):
    return pl.pallas_call(
        _inv_kernel,
        in_specs=[pl.BlockSpec(memory_space=pltpu.SMEM)],
        out_specs=pl.BlockSpec(memory_space=pltpu.SMEM),
        out_shape=jax.ShapeDtypeStruct((_T,), jnp.int32),
    )(dst)


def _sc_gather_body(x_hbm, src_hbm, xs_hbm, src_v, rows_v, sem):
    wid = lax.axis_index("s") * 2 + lax.axis_index("c")
    base = wid * _CHUNK
    pltpu.sync_copy(src_hbm.at[pl.ds(base, _CHUNK)], src_v)
    pltpu.async_copy(x_hbm.at[src_v], rows_v, sem).wait()
    pltpu.sync_copy(rows_v, xs_hbm.at[pl.ds(base, _CHUNK)])


def _sc_gather(x, src):
    mesh = plsc.VectorSubcoreMesh(core_axis_name="c", subcore_axis_name="s")
    f = functools.partial(
        pl.kernel, mesh=mesh,
        out_type=jax.ShapeDtypeStruct((_T, _D), jnp.float32),
        scratch_types=[
            pltpu.VMEM((_CHUNK,), jnp.int32),
            pltpu.VMEM((_CHUNK, _D), jnp.float32),
            pltpu.SemaphoreType.DMA,
        ],
    )(_sc_gather_body)
    return f(x, src)


_SLOTS = 6


def _seg_kernel(bounds_smem, xs_ref, bpre_ref, rb_ref, routert_ref, enct_ref,
                dec_ref, out_ref):
    t = pl.program_id(0)
    nt = _T // _ST
    e_lo = bounds_smem[128 - 2 * nt + t]
    e_hi = bounds_smem[128 - nt + t]
    xs = xs_ref[...]
    a = (xs - bpre_ref[...]).astype(jnp.bfloat16)
    grow = jax.lax.broadcasted_iota(jnp.int32, (_ST, 1), 0) + t * _ST

    logits = jax.lax.dot_general(xs - rb_ref[...], routert_ref[...], _DN_T,
                                 preferred_element_type=jnp.float32)
    m = jnp.max(logits, axis=-1, keepdims=True)
    ps_col = 1.0 / jnp.sum(jnp.exp(logits - m), axis=-1, keepdims=True)

    def enc_mm(e):
        return jax.lax.dot_general(a, enct_ref[e], _DN_T,
                                   preferred_element_type=jnp.float32)

    def mask_dec(e, lat, acc, valid):
        lat = jnp.maximum(lat, 0.0)
        seg_mask = (grow >= bounds_smem[e]) & (grow < bounds_smem[e + 1])
        lat = jnp.where(seg_mask & valid, lat, 0.0).astype(jnp.bfloat16)
        return acc + jnp.dot(lat, dec_ref[e], preferred_element_type=jnp.float32)

    es = [jnp.minimum(e_lo + i, _E - 1) for i in range(_SLOTS)]
    lats = [enc_mm(es[i]) for i in range(_SLOTS)]
    acc = jnp.zeros((_ST, _D), jnp.float32)
    for i in range(_SLOTS):
        acc = mask_dec(es[i], lats[i], acc, (e_lo + i) <= e_hi)

    acc = jax.lax.fori_loop(
        _SLOTS, e_hi - e_lo + 1,
        lambda i, s: mask_dec(e_lo + i, enc_mm(e_lo + i), s, True), acc)

    out_ref[...] = ps_col * acc + bpre_ref[...]


def _segment(xs, bounds, b_pre, router_b, router_t, enc_t, dec):
    grid_spec = pltpu.PrefetchScalarGridSpec(
        num_scalar_prefetch=1,
        grid=(_T // _ST,),
        in_specs=[
            pl.BlockSpec((_ST, _D), lambda t, b: (t, 0)),
            pl.BlockSpec((1, _D), lambda t, b: (0, 0)),
            pl.BlockSpec((1, _D), lambda t, b: (0, 0)),
            pl.BlockSpec((_E, _D), lambda t, b: (0, 0)),
            pl.BlockSpec((_E, _F, _D), lambda t, b: (0, 0, 0)),
            pl.BlockSpec((_E, _F, _D), lambda t, b: (0, 0, 0)),
        ],
        out_specs=pl.BlockSpec((_ST, _D), lambda t, b: (t, 0)),
    )
    return pl.pallas_call(
        _seg_kernel,
        grid_spec=grid_spec,
        out_shape=jax.ShapeDtypeStruct((_T, _D), jnp.float32),
    )(bounds, xs, b_pre.reshape(1, _D), router_b.reshape(1, _D), router_t,
      enc_t, dec)


def _sc_unsort_body(ys_hbm, dst_hbm, out_hbm, dst_v, rows_v, sem):
    wid = lax.axis_index("s") * 2 + lax.axis_index("c")
    base = wid * _CHUNK
    pltpu.sync_copy(dst_hbm.at[pl.ds(base, _CHUNK)], dst_v)
    pltpu.async_copy(ys_hbm.at[dst_v], rows_v, sem).wait()
    pltpu.sync_copy(rows_v, out_hbm.at[pl.ds(base, _CHUNK)])


def _sc_unsort(ys, dst):
    mesh = plsc.VectorSubcoreMesh(core_axis_name="c", subcore_axis_name="s")
    f = functools.partial(
        pl.kernel, mesh=mesh,
        out_type=jax.ShapeDtypeStruct((_T, _D), jnp.float32),
        scratch_types=[
            pltpu.VMEM((_CHUNK,), jnp.int32),
            pltpu.VMEM((_CHUNK, _D), jnp.float32),
            pltpu.SemaphoreType.DMA,
        ],
    )(_sc_unsort_body)
    return f(ys, dst)


def kernel(activations, b_pre, enc, dec, router_b, router):
    router_t = router.T
    enc_t = enc.transpose(0, 2, 1)
    dst, bounds = _plan(activations, router_b, router_t)
    src = _invert(dst)
    xs = _sc_gather(activations, src)
    ys = _segment(xs, bounds, b_pre, router_b, router_t,
                  enc_t.astype(jnp.bfloat16), dec.astype(jnp.bfloat16))
    return _sc_unsort(ys, dst)


# speedup vs baseline: 1.3003x; 1.1446x over previous
import functools

import jax
import jax.numpy as jnp
from jax import lax
from jax.experimental import pallas as pl
from jax.experimental.pallas import tpu as pltpu
from jax.experimental.pallas import tpu_sc as plsc

_T = 2048
_D = 768
_E = 64
_F = 64
_PT = 256
_ST = 128
_NW = 32
_CHUNK = _T // _NW

_DN_T = (((1,), (1,)), ((), ()))


def _plan_kernel(x_ref, rb_ref, routert_ref, dst_ref, bounds_ref):
    x = x_ref[...]
    logits = jax.lax.dot_general(x - rb_ref[...], routert_ref[...], _DN_T,
                                 preferred_element_type=jnp.float32)
    idx = jnp.argmax(logits, axis=-1)

    onehot = (jax.lax.broadcasted_iota(jnp.int32, (_T, _E), 1)
              == idx[:, None]).astype(jnp.float32)

    tri = (jax.lax.broadcasted_iota(jnp.int32, (_PT, _PT), 0)
           > jax.lax.broadcasted_iota(jnp.int32, (_PT, _PT), 1)
           ).astype(jnp.float32)
    ones_row = jnp.ones((1, _PT), dtype=jnp.float32)

    counts = jnp.zeros((1, _E), jnp.float32)
    rank_tiles = []
    for i in range(_T // _PT):
        blk = onehot[i * _PT:(i + 1) * _PT, :]
        rank_tiles.append(
            jnp.dot(tri, blk, preferred_element_type=jnp.float32) + counts)
        counts = counts + jnp.dot(ones_row, blk,
                                  preferred_element_type=jnp.float32)
    rank_all = jnp.concatenate(rank_tiles, axis=0)
    rank = jnp.sum(rank_all * onehot, axis=-1, keepdims=True)

    lt = (jax.lax.broadcasted_iota(jnp.int32, (_E, _E), 0)
          < jax.lax.broadcasted_iota(jnp.int32, (_E, _E), 1)).astype(jnp.float32)
    offsets = jnp.dot(counts, lt, preferred_element_type=jnp.float32)
    off_tok = jnp.sum(onehot * offsets, axis=-1, keepdims=True)

    cum_incl = offsets + counts
    cum_col = jnp.transpose(cum_incl, (1, 0))
    nt = _T // _ST
    slot_lo = (jax.lax.broadcasted_iota(jnp.int32, (1, nt), 1)
               * _ST).astype(jnp.float32)
    tile_lo = jnp.sum((cum_col <= slot_lo).astype(jnp.int32), axis=0,
                      keepdims=True)
    tile_hi = jnp.sum((cum_col <= slot_lo + float(_ST - 1)).astype(jnp.int32),
                      axis=0, keepdims=True)

    dst_ref[...] = (rank + off_tok).astype(jnp.int32).reshape(_T)
    bounds_ref[...] = jnp.concatenate(
        [offsets.astype(jnp.int32),
         jnp.full((1, 1), _T, jnp.int32),
         jnp.zeros((1, 127 - _E - 2 * nt), jnp.int32),
         tile_lo, tile_hi], axis=1).reshape(128)


def _plan(activations, router_b, router_t):
    return pl.pallas_call(
        _plan_kernel,
        in_specs=[
            pl.BlockSpec((_T, _D), lambda: (0, 0)),
            pl.BlockSpec((1, _D), lambda: (0, 0)),
            pl.BlockSpec((_E, _D), lambda: (0, 0)),
        ],
        out_specs=[
            pl.BlockSpec((_T,), lambda: (0,)),
            pl.BlockSpec((128,), lambda: (0,)),
        ],
        out_shape=[
            jax.ShapeDtypeStruct((_T,), jnp.int32),
            jax.ShapeDtypeStruct((128,), jnp.int32),
        ],
    )(activations, router_b.reshape(1, _D), router_t)


def _inv_kernel(dst_ref, src_ref):
    def body(t, carry):
        src_ref[dst_ref[t]] = t
        return carry

    jax.lax.fori_loop(0, _T, body, 0, unroll=8)


def _invert(dst):
    return pl.pallas_call(
        _inv_kernel,
        in_specs=[pl.BlockSpec(memory_space=pltpu.SMEM)],
        out_specs=pl.BlockSpec(memory_space=pltpu.SMEM),
        out_shape=jax.ShapeDtypeStruct((_T,), jnp.int32),
    )(dst)


def _sc_gather_body(x_hbm, src_hbm, xs_hbm, src_v, rows_v, sem):
    wid = lax.axis_index("s") * 2 + lax.axis_index("c")
    base = wid * _CHUNK
    pltpu.sync_copy(src_hbm.at[pl.ds(base, _CHUNK)], src_v)
    pltpu.async_copy(x_hbm.at[src_v], rows_v, sem).wait()
    pltpu.sync_copy(rows_v, xs_hbm.at[pl.ds(base, _CHUNK)])


def _sc_gather(x, src):
    mesh = plsc.VectorSubcoreMesh(core_axis_name="c", subcore_axis_name="s")
    f = functools.partial(
        pl.kernel, mesh=mesh,
        out_type=jax.ShapeDtypeStruct((_T, _D), jnp.float32),
        scratch_types=[
            pltpu.VMEM((_CHUNK,), jnp.int32),
            pltpu.VMEM((_CHUNK, _D), jnp.float32),
            pltpu.SemaphoreType.DMA,
        ],
    )(_sc_gather_body)
    return f(x, src)


_SLOTS = 6


def _seg_kernel(bounds_smem, xs_ref, bpre_ref, rb_ref, routert_ref, enct_ref,
                dec_ref, out_ref):
    t = pl.program_id(0)
    nt = _T // _ST
    e_lo = bounds_smem[128 - 2 * nt + t]
    e_hi = bounds_smem[128 - nt + t]
    xs = xs_ref[...]
    a = xs - bpre_ref[...]
    grow = jax.lax.broadcasted_iota(jnp.int32, (_ST, 1), 0) + t * _ST

    logits = jax.lax.dot_general(xs - rb_ref[...], routert_ref[...], _DN_T,
                                 preferred_element_type=jnp.float32)
    m = jnp.max(logits, axis=-1, keepdims=True)
    ps_col = 1.0 / jnp.sum(jnp.exp(logits - m), axis=-1, keepdims=True)

    def enc_mm(e):
        return jax.lax.dot_general(a, enct_ref[e], _DN_T,
                                   preferred_element_type=jnp.float32)

    def mask_dec(e, lat, acc, valid):
        lat = jnp.maximum(lat, 0.0)
        seg_mask = (grow >= bounds_smem[e]) & (grow < bounds_smem[e + 1])
        lat = jnp.where(seg_mask & valid, lat, 0.0)
        return acc + jnp.dot(lat, dec_ref[e], preferred_element_type=jnp.float32)

    es = [jnp.minimum(e_lo + i, _E - 1) for i in range(_SLOTS)]
    lats = [enc_mm(es[i]) for i in range(_SLOTS)]
    acc = jnp.zeros((_ST, _D), jnp.float32)
    for i in range(_SLOTS):
        acc = mask_dec(es[i], lats[i], acc, (e_lo + i) <= e_hi)

    acc = jax.lax.fori_loop(
        _SLOTS, e_hi - e_lo + 1,
        lambda i, s: mask_dec(e_lo + i, enc_mm(e_lo + i), s, True), acc)

    out_ref[...] = ps_col * acc + bpre_ref[...]


def _segment(xs, bounds, b_pre, router_b, router_t, enc_t, dec):
    grid_spec = pltpu.PrefetchScalarGridSpec(
        num_scalar_prefetch=1,
        grid=(_T // _ST,),
        in_specs=[
            pl.BlockSpec((_ST, _D), lambda t, b: (t, 0)),
            pl.BlockSpec((1, _D), lambda t, b: (0, 0)),
            pl.BlockSpec((1, _D), lambda t, b: (0, 0)),
            pl.BlockSpec((_E, _D), lambda t, b: (0, 0)),
            pl.BlockSpec((_E, _F, _D), lambda t, b: (0, 0, 0)),
            pl.BlockSpec((_E, _F, _D), lambda t, b: (0, 0, 0)),
        ],
        out_specs=pl.BlockSpec((_ST, _D), lambda t, b: (t, 0)),
    )
    return pl.pallas_call(
        _seg_kernel,
        grid_spec=grid_spec,
        out_shape=jax.ShapeDtypeStruct((_T, _D), jnp.float32),
    )(bounds, xs, b_pre.reshape(1, _D), router_b.reshape(1, _D), router_t,
      enc_t, dec)


def _sc_unsort_body(ys_hbm, dst_hbm, out_hbm, dst_v, rows_v, sem):
    wid = lax.axis_index("s") * 2 + lax.axis_index("c")
    base = wid * _CHUNK
    pltpu.sync_copy(dst_hbm.at[pl.ds(base, _CHUNK)], dst_v)
    pltpu.async_copy(ys_hbm.at[dst_v], rows_v, sem).wait()
    pltpu.sync_copy(rows_v, out_hbm.at[pl.ds(base, _CHUNK)])


def _sc_unsort(ys, dst):
    mesh = plsc.VectorSubcoreMesh(core_axis_name="c", subcore_axis_name="s")
    f = functools.partial(
        pl.kernel, mesh=mesh,
        out_type=jax.ShapeDtypeStruct((_T, _D), jnp.float32),
        scratch_types=[
            pltpu.VMEM((_CHUNK,), jnp.int32),
            pltpu.VMEM((_CHUNK, _D), jnp.float32),
            pltpu.SemaphoreType.DMA,
        ],
    )(_sc_unsort_body)
    return f(ys, dst)


def kernel(activations, b_pre, enc, dec, router_b, router):
    router_t = router.T
    enc_t = enc.transpose(0, 2, 1)
    dst, bounds = _plan(activations, router_b, router_t)
    src = _invert(dst)
    xs = _sc_gather(activations, src)
    ys = _segment(xs, bounds, b_pre, router_b, router_t, enc_t, dec)
    return _sc_unsort(ys, dst)


# speedup vs baseline: 1.3202x; 1.0153x over previous
import functools

import jax
import jax.numpy as jnp
from jax import lax
from jax.experimental import pallas as pl
from jax.experimental.pallas import tpu as pltpu
from jax.experimental.pallas import tpu_sc as plsc

_T = 2048
_D = 768
_E = 64
_F = 64
_PT = 256
_ST = 128
_NW = 32
_CHUNK = _T // _NW

_DN_T = (((1,), (1,)), ((), ()))


def _plan_kernel(x_ref, rb_ref, routert_ref, dst_ref, bounds_ref, src_ref,
                 dst_smem, sem):
    x = x_ref[...]
    logits = jax.lax.dot_general(x - rb_ref[...], routert_ref[...], _DN_T,
                                 preferred_element_type=jnp.float32)
    idx = jnp.argmax(logits, axis=-1)

    onehot = (jax.lax.broadcasted_iota(jnp.int32, (_T, _E), 1)
              == idx[:, None]).astype(jnp.float32)

    tri = (jax.lax.broadcasted_iota(jnp.int32, (_PT, _PT), 0)
           > jax.lax.broadcasted_iota(jnp.int32, (_PT, _PT), 1)
           ).astype(jnp.float32)
    ones_row = jnp.ones((1, _PT), dtype=jnp.float32)

    counts = jnp.zeros((1, _E), jnp.float32)
    rank_tiles = []
    for i in range(_T // _PT):
        blk = onehot[i * _PT:(i + 1) * _PT, :]
        rank_tiles.append(
            jnp.dot(tri, blk, preferred_element_type=jnp.float32) + counts)
        counts = counts + jnp.dot(ones_row, blk,
                                  preferred_element_type=jnp.float32)
    rank_all = jnp.concatenate(rank_tiles, axis=0)
    rank = jnp.sum(rank_all * onehot, axis=-1, keepdims=True)

    lt = (jax.lax.broadcasted_iota(jnp.int32, (_E, _E), 0)
          < jax.lax.broadcasted_iota(jnp.int32, (_E, _E), 1)).astype(jnp.float32)
    offsets = jnp.dot(counts, lt, preferred_element_type=jnp.float32)
    off_tok = jnp.sum(onehot * offsets, axis=-1, keepdims=True)

    cum_incl = offsets + counts
    cum_col = jnp.transpose(cum_incl, (1, 0))
    nt = _T // _ST
    slot_lo = (jax.lax.broadcasted_iota(jnp.int32, (1, nt), 1)
               * _ST).astype(jnp.float32)
    tile_lo = jnp.sum((cum_col <= slot_lo).astype(jnp.int32), axis=0,
                      keepdims=True)
    tile_hi = jnp.sum((cum_col <= slot_lo + float(_ST - 1)).astype(jnp.int32),
                      axis=0, keepdims=True)

    dst_ref[...] = (rank + off_tok).astype(jnp.int32).reshape(_T)
    bounds_ref[...] = jnp.concatenate(
        [offsets.astype(jnp.int32),
         jnp.full((1, 1), _T, jnp.int32),
         jnp.zeros((1, 127 - _E - 2 * nt), jnp.int32),
         tile_lo, tile_hi], axis=1).reshape(128)

    pltpu.make_async_copy(dst_ref, dst_smem, sem).start()
    pltpu.make_async_copy(dst_ref, dst_smem, sem).wait()

    def body(t, carry):
        src_ref[dst_smem[t]] = t
        return carry

    jax.lax.fori_loop(0, _T, body, 0, unroll=8)


def _plan(activations, router_b, router_t):
    return pl.pallas_call(
        _plan_kernel,
        in_specs=[
            pl.BlockSpec((_T, _D), lambda: (0, 0)),
            pl.BlockSpec((1, _D), lambda: (0, 0)),
            pl.BlockSpec((_E, _D), lambda: (0, 0)),
        ],
        out_specs=[
            pl.BlockSpec((_T,), lambda: (0,)),
            pl.BlockSpec((128,), lambda: (0,)),
            pl.BlockSpec(memory_space=pltpu.SMEM),
        ],
        out_shape=[
            jax.ShapeDtypeStruct((_T,), jnp.int32),
            jax.ShapeDtypeStruct((128,), jnp.int32),
            jax.ShapeDtypeStruct((_T,), jnp.int32),
        ],
        scratch_shapes=[
            pltpu.SMEM((_T,), jnp.int32),
            pltpu.SemaphoreType.DMA,
        ],
    )(activations, router_b.reshape(1, _D), router_t)


def _sc_gather_body(x_hbm, src_hbm, xs_hbm, src_v, rows_v, sem):
    wid = lax.axis_index("s") * 2 + lax.axis_index("c")
    base = wid * _CHUNK
    pltpu.sync_copy(src_hbm.at[pl.ds(base, _CHUNK)], src_v)
    pltpu.async_copy(x_hbm.at[src_v], rows_v, sem).wait()
    pltpu.sync_copy(rows_v, xs_hbm.at[pl.ds(base, _CHUNK)])


def _sc_gather(x, src):
    mesh = plsc.VectorSubcoreMesh(core_axis_name="c", subcore_axis_name="s")
    f = functools.partial(
        pl.kernel, mesh=mesh,
        out_type=jax.ShapeDtypeStruct((_T, _D), jnp.float32),
        scratch_types=[
            pltpu.VMEM((_CHUNK,), jnp.int32),
            pltpu.VMEM((_CHUNK, _D), jnp.float32),
            pltpu.SemaphoreType.DMA,
        ],
    )(_sc_gather_body)
    return f(x, src)


_SLOTS = 6


def _seg_kernel(bounds_smem, xs_ref, bpre_ref, rb_ref, routert_ref, enct_ref,
                dec_ref, out_ref):
    t = pl.program_id(0)
    nt = _T // _ST
    e_lo = bounds_smem[128 - 2 * nt + t]
    e_hi = bounds_smem[128 - nt + t]
    xs = xs_ref[...]
    a = xs - bpre_ref[...]
    grow = jax.lax.broadcasted_iota(jnp.int32, (_ST, 1), 0) + t * _ST

    logits = jax.lax.dot_general(xs - rb_ref[...], routert_ref[...], _DN_T,
                                 preferred_element_type=jnp.float32)
    m = jnp.max(logits, axis=-1, keepdims=True)
    ps_col = 1.0 / jnp.sum(jnp.exp(logits - m), axis=-1, keepdims=True)

    def enc_mm(e):
        return jax.lax.dot_general(a, enct_ref[e], _DN_T,
                                   preferred_element_type=jnp.float32)

    def mask_dec(e, lat, acc, valid):
        lat = jnp.maximum(lat, 0.0)
        seg_mask = (grow >= bounds_smem[e]) & (grow < bounds_smem[e + 1])
        lat = jnp.where(seg_mask & valid, lat, 0.0)
        return acc + jnp.dot(lat, dec_ref[e], preferred_element_type=jnp.float32)

    es = [jnp.minimum(e_lo + i, _E - 1) for i in range(_SLOTS)]
    lats = [enc_mm(es[i]) for i in range(_SLOTS)]
    acc = jnp.zeros((_ST, _D), jnp.float32)
    for i in range(_SLOTS):
        acc = mask_dec(es[i], lats[i], acc, (e_lo + i) <= e_hi)

    acc = jax.lax.fori_loop(
        _SLOTS, e_hi - e_lo + 1,
        lambda i, s: mask_dec(e_lo + i, enc_mm(e_lo + i), s, True), acc)

    out_ref[...] = ps_col * acc + bpre_ref[...]


def _segment(xs, bounds, b_pre, router_b, router_t, enc_t, dec):
    grid_spec = pltpu.PrefetchScalarGridSpec(
        num_scalar_prefetch=1,
        grid=(_T // _ST,),
        in_specs=[
            pl.BlockSpec((_ST, _D), lambda t, b: (t, 0)),
            pl.BlockSpec((1, _D), lambda t, b: (0, 0)),
            pl.BlockSpec((1, _D), lambda t, b: (0, 0)),
            pl.BlockSpec((_E, _D), lambda t, b: (0, 0)),
            pl.BlockSpec((_E, _F, _D), lambda t, b: (0, 0, 0)),
            pl.BlockSpec((_E, _F, _D), lambda t, b: (0, 0, 0)),
        ],
        out_specs=pl.BlockSpec((_ST, _D), lambda t, b: (t, 0)),
    )
    return pl.pallas_call(
        _seg_kernel,
        grid_spec=grid_spec,
        out_shape=jax.ShapeDtypeStruct((_T, _D), jnp.float32),
    )(bounds, xs, b_pre.reshape(1, _D), router_b.reshape(1, _D), router_t,
      enc_t, dec)


def _sc_unsort_body(ys_hbm, dst_hbm, out_hbm, dst_v, rows_v, sem):
    wid = lax.axis_index("s") * 2 + lax.axis_index("c")
    base = wid * _CHUNK
    pltpu.sync_copy(dst_hbm.at[pl.ds(base, _CHUNK)], dst_v)
    pltpu.async_copy(ys_hbm.at[dst_v], rows_v, sem).wait()
    pltpu.sync_copy(rows_v, out_hbm.at[pl.ds(base, _CHUNK)])


def _sc_unsort(ys, dst):
    mesh = plsc.VectorSubcoreMesh(core_axis_name="c", subcore_axis_name="s")
    f = functools.partial(
        pl.kernel, mesh=mesh,
        out_type=jax.ShapeDtypeStruct((_T, _D), jnp.float32),
        scratch_types=[
            pltpu.VMEM((_CHUNK,), jnp.int32),
            pltpu.VMEM((_CHUNK, _D), jnp.float32),
            pltpu.SemaphoreType.DMA,
        ],
    )(_sc_unsort_body)
    return f(ys, dst)


def kernel(activations, b_pre, enc, dec, router_b, router):
    router_t = router.T
    enc_t = enc.transpose(0, 2, 1)
    dst, bounds, src = _plan(activations, router_b, router_t)
    xs = _sc_gather(activations, src)
    ys = _segment(xs, bounds, b_pre, router_b, router_t, enc_t, dec)
    return _sc_unsort(ys, dst)


# speedup vs baseline: 1.3825x; 1.0472x over previous
import functools

import jax
import jax.numpy as jnp
from jax import lax
from jax.experimental import pallas as pl
from jax.experimental.pallas import tpu as pltpu
from jax.experimental.pallas import tpu_sc as plsc

_T = 2048
_D = 768
_E = 64
_F = 64
_PT = 256
_ST = 128
_NW = 32
_CHUNK = _T // _NW

_DN_T = (((1,), (1,)), ((), ()))


def _plan_kernel(x_ref, rb_ref, routert_ref, dst_ref, bounds_ref, src_ref,
                 dst_smem, sem):
    x = x_ref[...]
    logits = jax.lax.dot_general(x - rb_ref[...], routert_ref[...], _DN_T,
                                 preferred_element_type=jnp.float32)
    idx = jnp.argmax(logits, axis=-1)

    onehot = (jax.lax.broadcasted_iota(jnp.int32, (_T, _E), 1)
              == idx[:, None]).astype(jnp.float32)

    tri = (jax.lax.broadcasted_iota(jnp.int32, (_PT, _PT), 0)
           > jax.lax.broadcasted_iota(jnp.int32, (_PT, _PT), 1)
           ).astype(jnp.float32)
    ones_row = jnp.ones((1, _PT), dtype=jnp.float32)

    counts = jnp.zeros((1, _E), jnp.float32)
    rank_tiles = []
    for i in range(_T // _PT):
        blk = onehot[i * _PT:(i + 1) * _PT, :]
        rank_tiles.append(
            jnp.dot(tri, blk, preferred_element_type=jnp.float32) + counts)
        counts = counts + jnp.dot(ones_row, blk,
                                  preferred_element_type=jnp.float32)
    rank_all = jnp.concatenate(rank_tiles, axis=0)
    rank = jnp.sum(rank_all * onehot, axis=-1, keepdims=True)

    lt = (jax.lax.broadcasted_iota(jnp.int32, (_E, _E), 0)
          < jax.lax.broadcasted_iota(jnp.int32, (_E, _E), 1)).astype(jnp.float32)
    offsets = jnp.dot(counts, lt, preferred_element_type=jnp.float32)
    off_tok = jnp.sum(onehot * offsets, axis=-1, keepdims=True)

    cum_incl = offsets + counts
    cum_col = jnp.transpose(cum_incl, (1, 0))
    nt = _T // _ST
    slot_lo = (jax.lax.broadcasted_iota(jnp.int32, (1, nt), 1)
               * _ST).astype(jnp.float32)
    tile_lo = jnp.sum((cum_col <= slot_lo).astype(jnp.int32), axis=0,
                      keepdims=True)
    tile_hi = jnp.sum((cum_col <= slot_lo + float(_ST - 1)).astype(jnp.int32),
                      axis=0, keepdims=True)

    dst_ref[...] = (rank + off_tok).astype(jnp.int32).reshape(_T)
    bounds_ref[...] = jnp.concatenate(
        [offsets.astype(jnp.int32),
         jnp.full((1, 1), _T, jnp.int32),
         jnp.zeros((1, 127 - _E - 2 * nt), jnp.int32),
         tile_lo, tile_hi], axis=1).reshape(128)

    pltpu.make_async_copy(dst_ref, dst_smem, sem).start()
    pltpu.make_async_copy(dst_ref, dst_smem, sem).wait()

    def body(t, carry):
        src_ref[dst_smem[t]] = t
        return carry

    jax.lax.fori_loop(0, _T, body, 0, unroll=8)


def _plan(activations, router_b, router_t):
    return pl.pallas_call(
        _plan_kernel,
        in_specs=[
            pl.BlockSpec((_T, _D), lambda: (0, 0)),
            pl.BlockSpec((1, _D), lambda: (0, 0)),
            pl.BlockSpec((_E, _D), lambda: (0, 0)),
        ],
        out_specs=[
            pl.BlockSpec((_T,), lambda: (0,)),
            pl.BlockSpec((128,), lambda: (0,)),
            pl.BlockSpec(memory_space=pltpu.SMEM),
        ],
        out_shape=[
            jax.ShapeDtypeStruct((_T,), jnp.int32),
            jax.ShapeDtypeStruct((128,), jnp.int32),
            jax.ShapeDtypeStruct((_T,), jnp.int32),
        ],
        scratch_shapes=[
            pltpu.SMEM((_T,), jnp.int32),
            pltpu.SemaphoreType.DMA,
        ],
    )(activations, router_b.reshape(1, _D), router_t)


def _sc_gather_body(x_hbm, src_hbm, xs_hbm, src_v, rows_v, sem):
    wid = lax.axis_index("s") * 2 + lax.axis_index("c")
    base = wid * _CHUNK
    pltpu.sync_copy(src_hbm.at[pl.ds(base, _CHUNK)], src_v)
    pltpu.async_copy(x_hbm.at[src_v], rows_v, sem).wait()
    pltpu.sync_copy(rows_v, xs_hbm.at[pl.ds(base, _CHUNK)])


def _sc_gather(x, src):
    mesh = plsc.VectorSubcoreMesh(core_axis_name="c", subcore_axis_name="s")
    f = functools.partial(
        pl.kernel, mesh=mesh,
        out_type=jax.ShapeDtypeStruct((_T, _D), jnp.float32),
        scratch_types=[
            pltpu.VMEM((_CHUNK,), jnp.int32),
            pltpu.VMEM((_CHUNK, _D), jnp.float32),
            pltpu.SemaphoreType.DMA,
        ],
    )(_sc_gather_body)
    return f(x, src)


_SLOTS = 6


def _seg_kernel(bounds_smem, xs_ref, bpre_ref, rb_ref, routert_ref, enct_ref,
                dec_ref, out_ref):
    g = pl.program_id(0)
    nt = _T // _ST
    xs = xs_ref[...]
    a_full = xs - bpre_ref[...]

    logits = jax.lax.dot_general(xs - rb_ref[...], routert_ref[...], _DN_T,
                                 preferred_element_type=jnp.float32)
    m = jnp.max(logits, axis=-1, keepdims=True)
    ps_col = 1.0 / jnp.sum(jnp.exp(logits - m), axis=-1, keepdims=True)

    for k in range(2):
        t = g * 2 + k
        e_lo = bounds_smem[128 - 2 * nt + t]
        e_hi = bounds_smem[128 - nt + t]
        a = a_full[k * _ST:(k + 1) * _ST, :]
        grow = jax.lax.broadcasted_iota(jnp.int32, (_ST, 1), 0) + t * _ST

        def enc_mm(e):
            return jax.lax.dot_general(a, enct_ref[e], _DN_T,
                                       preferred_element_type=jnp.float32)

        def mask_dec(e, lat, acc, valid):
            lat = jnp.maximum(lat, 0.0)
            seg_mask = (grow >= bounds_smem[e]) & (grow < bounds_smem[e + 1])
            lat = jnp.where(seg_mask & valid, lat, 0.0)
            return acc + jnp.dot(lat, dec_ref[e],
                                 preferred_element_type=jnp.float32)

        es = [jnp.minimum(e_lo + i, _E - 1) for i in range(_SLOTS)]
        lats = [enc_mm(es[i]) for i in range(_SLOTS)]
        acc = jnp.zeros((_ST, _D), jnp.float32)
        for i in range(_SLOTS):
            acc = mask_dec(es[i], lats[i], acc, (e_lo + i) <= e_hi)

        acc = jax.lax.fori_loop(
            _SLOTS, e_hi - e_lo + 1,
            lambda i, s: mask_dec(e_lo + i, enc_mm(e_lo + i), s, True), acc)

        out_ref[k * _ST:(k + 1) * _ST, :] = (
            ps_col[k * _ST:(k + 1) * _ST, :] * acc + bpre_ref[...])


def _segment(xs, bounds, b_pre, router_b, router_t, enc_t, dec):
    grid_spec = pltpu.PrefetchScalarGridSpec(
        num_scalar_prefetch=1,
        grid=(_T // (2 * _ST),),
        in_specs=[
            pl.BlockSpec((2 * _ST, _D), lambda t, b: (t, 0)),
            pl.BlockSpec((1, _D), lambda t, b: (0, 0)),
            pl.BlockSpec((1, _D), lambda t, b: (0, 0)),
            pl.BlockSpec((_E, _D), lambda t, b: (0, 0)),
            pl.BlockSpec((_E, _F, _D), lambda t, b: (0, 0, 0)),
            pl.BlockSpec((_E, _F, _D), lambda t, b: (0, 0, 0)),
        ],
        out_specs=pl.BlockSpec((2 * _ST, _D), lambda t, b: (t, 0)),
    )
    return pl.pallas_call(
        _seg_kernel,
        grid_spec=grid_spec,
        out_shape=jax.ShapeDtypeStruct((_T, _D), jnp.float32),
    )(bounds, xs, b_pre.reshape(1, _D), router_b.reshape(1, _D), router_t,
      enc_t, dec)


def _sc_unsort_body(ys_hbm, dst_hbm, out_hbm, dst_v, rows_v, sem):
    wid = lax.axis_index("s") * 2 + lax.axis_index("c")
    base = wid * _CHUNK
    pltpu.sync_copy(dst_hbm.at[pl.ds(base, _CHUNK)], dst_v)
    pltpu.async_copy(ys_hbm.at[dst_v], rows_v, sem).wait()
    pltpu.sync_copy(rows_v, out_hbm.at[pl.ds(base, _CHUNK)])


def _sc_unsort(ys, dst):
    mesh = plsc.VectorSubcoreMesh(core_axis_name="c", subcore_axis_name="s")
    f = functools.partial(
        pl.kernel, mesh=mesh,
        out_type=jax.ShapeDtypeStruct((_T, _D), jnp.float32),
        scratch_types=[
            pltpu.VMEM((_CHUNK,), jnp.int32),
            pltpu.VMEM((_CHUNK, _D), jnp.float32),
            pltpu.SemaphoreType.DMA,
        ],
    )(_sc_unsort_body)
    return f(ys, dst)


def kernel(activations, b_pre, enc, dec, router_b, router):
    router_t = router.T
    enc_t = enc.transpose(0, 2, 1)
    dst, bounds, src = _plan(activations, router_b, router_t)
    xs = _sc_gather(activations, src)
    ys = _segment(xs, bounds, b_pre, router_b, router_t, enc_t, dec)
    return _sc_unsort(ys, dst)
